# baseline (speedup 1.0000x reference)
import jax
import jax.numpy as jnp
from jax.experimental import pallas as pl
from jax.experimental.pallas import tpu as pltpu

E = 8
TOP_K = 2
H = 1024
FF = 3584
FF_T = 896
N_T = FF // FF_T
H_T = 256
N_H = H // H_T


def _moe_step(x_ref, gate_ref, w1_ref, w3_ref, w2_ref, out_ref,
              coef_ref, t_ref, acc_ref):
    e = pl.program_id(0)
    j = pl.program_id(1)

    @pl.when((e == 0) & (j == 0))
    def _router():
        x = x_ref[...]
        logits = jnp.dot(
            x.astype(jnp.bfloat16),
            gate_ref[...].astype(jnp.bfloat16).T,
            preferred_element_type=jnp.float32,
        )
        m = jnp.max(logits, axis=1, keepdims=True)
        p = jnp.exp(logits - m)
        p = p / jnp.sum(p, axis=1, keepdims=True)
        idx = jax.lax.broadcasted_iota(jnp.int32, p.shape, 1)
        v1 = jnp.max(p, axis=1, keepdims=True)
        i1 = jnp.min(jnp.where(p == v1, idx, E), axis=1, keepdims=True)
        p2 = jnp.where(idx == i1, -jnp.inf, p)
        v2 = jnp.max(p2, axis=1, keepdims=True)
        i2 = jnp.min(jnp.where(p2 == v2, idx, E), axis=1, keepdims=True)
        sel = jnp.where(idx == i1, v1, 0.0) + jnp.where(idx == i2, v2, 0.0)
        coef_ref[...] = sel / (v1 + v2)

    @pl.when(j < N_T)
    def _phase_a():
        xb = x_ref[...].astype(jnp.bfloat16)
        a = jnp.dot(xb, w1_ref[0].astype(jnp.bfloat16).T,
                    preferred_element_type=jnp.float32)
        b = jnp.dot(xb, w3_ref[0].astype(jnp.bfloat16).T,
                    preferred_element_type=jnp.float32)
        t = (a * jax.nn.sigmoid(a)) * b
        idx = jax.lax.broadcasted_iota(jnp.int32, coef_ref.shape, 1)
        coef_col = jnp.sum(
            jnp.where(idx == e, coef_ref[...], 0.0), axis=1, keepdims=True
        )
        t_ref[j] = (t * coef_col).astype(jnp.bfloat16)

    @pl.when(j >= N_T)
    def _phase_b():
        jj = j - N_T
        w2b = w2_ref[0].astype(jnp.bfloat16)
        contrib = jnp.dot(t_ref[0], w2b[:, 0:FF_T].T,
                          preferred_element_type=jnp.float32)
        for k in range(1, N_T):
            contrib += jnp.dot(t_ref[k], w2b[:, k * FF_T:(k + 1) * FF_T].T,
                               preferred_element_type=jnp.float32)

        @pl.when(e == 0)
        def _():
            acc_ref[jj] = contrib

        @pl.when(e > 0)
        def _():
            acc_ref[jj] += contrib

    @pl.when((e == E - 1) & (j == N_T + N_H - 1))
    def _writeback():
        for k in range(N_H):
            out_ref[:, k * H_T:(k + 1) * H_T] = acc_ref[k]


def kernel(hidden_states, gate_w, w1, w3, w2, prefetch_expert_idx):
    b, s, h = hidden_states.shape
    x = hidden_states.reshape(-1, h)
    n = x.shape[0]

    out = pl.pallas_call(
        _moe_step,
        grid=(E, N_T + N_H),
        in_specs=[
            pl.BlockSpec((n, H), lambda e, j: (0, 0)),
            pl.BlockSpec((E, H), lambda e, j: (0, 0)),
            pl.BlockSpec((1, FF_T, H), lambda e, j: (e, jnp.minimum(j, N_T - 1), 0)),
            pl.BlockSpec((1, FF_T, H), lambda e, j: (e, jnp.minimum(j, N_T - 1), 0)),
            pl.BlockSpec((1, H_T, FF), lambda e, j: (e, jnp.maximum(j - N_T, 0), 0)),
        ],
        out_specs=pl.BlockSpec((n, H), lambda e, j: (0, 0)),
        out_shape=jax.ShapeDtypeStruct((n, H), jnp.float32),
        scratch_shapes=[
            pltpu.VMEM((n, E), jnp.float32),
            pltpu.VMEM((N_T, n, FF_T), jnp.bfloat16),
            pltpu.VMEM((N_H, n, H_T), jnp.float32),
        ],
    )(x, gate_w, w1, w3, w2)
    return out.reshape(b, s, h)


# speedup vs baseline: 1.0182x; 1.0182x over previous
import dataclasses

import jax
import jax.numpy as jnp
from jax.experimental import pallas as pl
from jax.experimental.pallas import tpu as pltpu
from jax.experimental.pallas import tpu_sc as plsc

E = 8
TOP_K = 2
H = 1024
FF = 3584
FF_T = 1792
N_T = FF // FF_T

_LANES = 16
_CHUNKS = H // _LANES


def _sc_router(x, gate_w):
    n = x.shape[0]
    mesh = plsc.VectorSubcoreMesh(core_axis_name="c", subcore_axis_name="s")
    cp = pltpu.CompilerParams()
    if "needs_layout_passes" in pltpu.CompilerParams.__dataclass_fields__:
        cp = dataclasses.replace(cp, needs_layout_passes=False)

    @pl.kernel(
        compiler_params=cp,
        out_type=jax.ShapeDtypeStruct((n, _LANES), jnp.float32),
        mesh=mesh,
        scratch_types=[
            pltpu.VMEM((H,), jnp.float32),
            pltpu.VMEM((E, H), jnp.float32),
            pltpu.VMEM((E, _LANES), jnp.float32),
            pltpu.VMEM((_LANES,), jnp.float32),
        ],
    )
    def _router(x_hbm, g_hbm, o_hbm, xrow, gmat, acc, crow):
        c = jax.lax.axis_index("c")
        s = jax.lax.axis_index("s")
        t = c * 16 + s
        pltpu.sync_copy(x_hbm.at[t], xrow)
        pltpu.sync_copy(g_hbm, gmat)

        for e in range(E):
            acc[e, :] = jnp.zeros((_LANES,), jnp.float32)

        def bf16_round(v):
            u = plsc.bitcast(v, jnp.int32)
            u = (u + jnp.int32(0x7FFF) + ((u >> 16) & 1)) & jnp.int32(-65536)
            return plsc.bitcast(u, jnp.float32)

        @pl.loop(0, _CHUNKS)
        def _(ci):
            sl = pl.ds(ci * _LANES, _LANES)
            xc = bf16_round(xrow[sl])
            for e in range(E):
                gc = bf16_round(gmat[e, sl])
                acc[e, :] += xc * gc

        lane = jax.lax.iota(jnp.int32, _LANES)
        logits = jnp.full((_LANES,), -1e30, jnp.float32)
        for e in range(E):
            logits = jnp.where(lane == e, jnp.sum(acc[e, :]), logits)

        m = jnp.max(logits)
        p = jnp.exp(logits - m)
        p = p / jnp.sum(p)
        v1 = jnp.max(p)
        i1 = jnp.min(jnp.where(p == v1, lane, _LANES))
        p2 = jnp.where(lane == i1, -1.0, p)
        v2 = jnp.max(p2)
        i2 = jnp.min(jnp.where(p2 == v2, lane, _LANES))
        sel = jnp.where(lane == i1, v1, 0.0) + jnp.where(lane == i2, v2, 0.0)
        crow[...] = sel / (v1 + v2)
        pltpu.sync_copy(crow, o_hbm.at[t])

    return _router(x, gate_w)


def _moe_step(x_ref, coef_ref, w1_ref, w3_ref, w2_ref, out_ref):
    e = pl.program_id(0)
    j = pl.program_id(1)

    @pl.when((e == 0) & (j == 0))
    def _init():
        out_ref[...] = jnp.zeros_like(out_ref)

    xb = x_ref[...].astype(jnp.bfloat16)
    w1b = w1_ref[0].astype(jnp.bfloat16)
    w3b = w3_ref[0].astype(jnp.bfloat16)
    a = jnp.dot(xb, w1b.T, preferred_element_type=jnp.float32)
    b = jnp.dot(xb, w3b.T, preferred_element_type=jnp.float32)
    t = (a * jax.nn.sigmoid(a)) * b

    idx = jax.lax.broadcasted_iota(jnp.int32, coef_ref.shape, 1)
    coef_col = jnp.sum(
        jnp.where(idx == e, coef_ref[...], 0.0), axis=1, keepdims=True
    )
    t = t * coef_col

    w2b = w2_ref[0].astype(jnp.bfloat16)
    out_ref[...] += jnp.dot(
        t.astype(jnp.bfloat16), w2b.T, preferred_element_type=jnp.float32
    )


def kernel(hidden_states, gate_w, w1, w3, w2, prefetch_expert_idx):
    b, s, h = hidden_states.shape
    x = hidden_states.reshape(-1, h)
    n = x.shape[0]

    coef = _sc_router(x, gate_w)

    out = pl.pallas_call(
        _moe_step,
        grid=(E, N_T),
        in_specs=[
            pl.BlockSpec((n, H), lambda e, j: (0, 0)),
            pl.BlockSpec((n, _LANES), lambda e, j: (0, 0)),
            pl.BlockSpec((1, FF_T, H), lambda e, j: (e, j, 0)),
            pl.BlockSpec((1, FF_T, H), lambda e, j: (e, j, 0)),
            pl.BlockSpec((1, H, FF_T), lambda e, j: (e, 0, j)),
        ],
        out_specs=pl.BlockSpec((n, H), lambda e, j: (0, 0)),
        out_shape=jax.ShapeDtypeStruct((n, H), jnp.float32),
    )(x, coef, w1, w3, w2)
    return out.reshape(b, s, h)


# speedup vs baseline: 1.2566x; 1.2341x over previous
import jax
import jax.numpy as jnp
from jax.experimental import pallas as pl
from jax.experimental.pallas import tpu as pltpu

E = 8
TOP_K = 2
H = 1024
FF = 3584
FF_T = 896
N_T = FF // FF_T


def _moe_step(x_ref, gate_ref, w1_ref, w3_ref, w2_ref, out_ref, coef_ref):
    e = pl.program_id(0)
    j = pl.program_id(1)

    @pl.when((e == 0) & (j == 0))
    def _router_and_init():
        x = x_ref[...]
        logits = jnp.dot(
            x.astype(jnp.bfloat16),
            gate_ref[...].astype(jnp.bfloat16).T,
            preferred_element_type=jnp.float32,
        )
        m = jnp.max(logits, axis=1, keepdims=True)
        p = jnp.exp(logits - m)
        p = p / jnp.sum(p, axis=1, keepdims=True)
        idx = jax.lax.broadcasted_iota(jnp.int32, p.shape, 1)
        v1 = jnp.max(p, axis=1, keepdims=True)
        i1 = jnp.min(jnp.where(p == v1, idx, E), axis=1, keepdims=True)
        p2 = jnp.where(idx == i1, -jnp.inf, p)
        v2 = jnp.max(p2, axis=1, keepdims=True)
        i2 = jnp.min(jnp.where(p2 == v2, idx, E), axis=1, keepdims=True)
        sel = jnp.where(idx == i1, v1, 0.0) + jnp.where(idx == i2, v2, 0.0)
        coef_ref[...] = sel / (v1 + v2)
        out_ref[...] = jnp.zeros_like(out_ref)

    xb = x_ref[...].astype(jnp.bfloat16)
    w1b = w1_ref[0].astype(jnp.bfloat16)
    w3b = w3_ref[0].astype(jnp.bfloat16)
    a = jnp.dot(xb, w1b.T, preferred_element_type=jnp.float32)
    b = jnp.dot(xb, w3b.T, preferred_element_type=jnp.float32)
    t = (a * jax.nn.sigmoid(a)) * b

    idx = jax.lax.broadcasted_iota(jnp.int32, coef_ref.shape, 1)
    coef_col = jnp.sum(
        jnp.where(idx == e, coef_ref[...], 0.0), axis=1, keepdims=True
    )
    t = t * coef_col

    w2b = w2_ref[0].astype(jnp.bfloat16)
    out_ref[...] += jnp.dot(
        t.astype(jnp.bfloat16), w2b.T, preferred_element_type=jnp.float32
    )


def kernel(hidden_states, gate_w, w1, w3, w2, prefetch_expert_idx):
    b, s, h = hidden_states.shape
    x = hidden_states.reshape(-1, h)
    n = x.shape[0]

    out = pl.pallas_call(
        _moe_step,
        grid=(E, N_T),
        in_specs=[
            pl.BlockSpec((n, H), lambda e, j: (0, 0)),
            pl.BlockSpec((E, H), lambda e, j: (0, 0)),
            pl.BlockSpec((1, FF_T, H), lambda e, j: (e, j, 0)),
            pl.BlockSpec((1, FF_T, H), lambda e, j: (e, j, 0)),
            pl.BlockSpec((1, H, FF_T), lambda e, j: (e, 0, j)),
        ],
        out_specs=pl.BlockSpec((n, H), lambda e, j: (0, 0)),
        out_shape=jax.ShapeDtypeStruct((n, H), jnp.float32),
        scratch_shapes=[pltpu.VMEM((n, E), jnp.float32)],
    )(x, gate_w, w1, w3, w2)
    return out.reshape(b, s, h)
